# baseline (speedup 1.0000x reference)
import functools

import jax
import jax.numpy as jnp
from jax import lax
from jax.experimental import pallas as pl
from jax.experimental.pallas import tpu as pltpu
from jax.experimental.pallas import tpu_sc as plsc

V = 1000000
D = 64
R = 16
B = 16384
L = 20

NC, NS = 2, 16
NW = NC * NS
N = B * L
PW = N // NW
CH = 320
NCH = PW // CH
SUB = CH // 64
GRP = CH // L
VPG = L * D // 16
IPW = PW // 64
BPC = CH // L
BPW = B // NW


def _adap_body(idx0_ref, wd_ref, bd_ref, wu_ref, bu_ref, table_ref,
               out_ref, emb_ref, sem):
    for i in range(L):
        pltpu.make_async_copy(
            table_ref.at[pl.ds(idx0_ref[i], 1)], emb_ref.at[pl.ds(i, 1)], sem
        ).start()
    for i in range(L):
        pltpu.make_async_copy(
            table_ref.at[pl.ds(idx0_ref[i], 1)], emb_ref.at[pl.ds(i, 1)], sem
        ).wait()
    h = emb_ref[...]
    mid = jnp.maximum(
        jnp.dot(h, wd_ref[...], preferred_element_type=jnp.float32)
        + bd_ref[...], 0.0)
    out_ref[...] = (h
                    + jnp.dot(mid, wu_ref[...],
                              preferred_element_type=jnp.float32)
                    + bu_ref[...])


_adapter_call = pl.pallas_call(
    _adap_body,
    out_shape=jax.ShapeDtypeStruct((L, D), jnp.float32),
    in_specs=[
        pl.BlockSpec(memory_space=pltpu.SMEM),
        pl.BlockSpec(memory_space=pltpu.VMEM),
        pl.BlockSpec(memory_space=pltpu.VMEM),
        pl.BlockSpec(memory_space=pltpu.VMEM),
        pl.BlockSpec(memory_space=pltpu.VMEM),
        pl.BlockSpec(memory_space=pltpu.MemorySpace.HBM),
    ],
    out_specs=pl.BlockSpec(memory_space=pltpu.VMEM),
    scratch_shapes=[pltpu.VMEM((L, D), jnp.float32), pltpu.SemaphoreType.DMA],
)


def _sc_body(table2, idxr, adap, out,
             idx_v, idx2_v, offs_v, adap_v, rows_v, gs0, gs1, ss0, ss1):
    wid = lax.axis_index("s") * NC + lax.axis_index("c")
    ibase = wid * IPW
    bbase = wid * BPW

    pltpu.sync_copy(idxr.at[pl.ds(ibase, IPW)], idx_v)
    pltpu.sync_copy(adap, adap_v)

    one = jnp.int32(1)
    six = jnp.int32(6)

    @pl.loop(0, IPW)
    def _(j):
        for k in range(4):
            x = idx_v[j, pl.ds(k * 16, 16)]
            idx2_v[j, pl.ds(k * 16, 16)] = lax.shift_right_logical(x, one)
            offs_v[pl.ds(j * 64 + k * 16, 16)] = lax.shift_left(
                lax.bitwise_and(x, one), six)

    gsems = (gs0, gs1)
    ssems = (ss0, ss1)

    def start_gather(c, buf):
        for j in range(SUB):
            pltpu.async_copy(
                table2.at[idx2_v.at[c * SUB + j]],
                rows_v.at[buf, pl.ds(j * 64, 64), :],
                gsems[buf])

    def wait_gather(buf):
        for j in range(SUB):
            pltpu.make_async_copy(
                table2.at[idx2_v.at[j]],
                rows_v.at[buf, pl.ds(j * 64, 64), :],
                gsems[buf]).wait()

    def add_chunk(c, buf):
        fbase = c * CH

        @pl.loop(0, GRP)
        def _(g):
            rbase = g * L
            for rr in range(L):
                r = rbase + rr
                soff = offs_v[pl.ds(fbase + r, 16)][0]
                for k in range(4):
                    rows_v[buf, r, pl.ds(k * 16, 16)] = (
                        rows_v[buf, r, pl.ds(soff + k * 16, 16)]
                        + adap_v[rr * 4 + k, :])

    def start_store(c, buf):
        for k in range(BPC):
            pltpu.async_copy(
                rows_v.at[buf, pl.ds(k * L, L), pl.ds(0, 64)],
                out.at[bbase + c * BPC + k],
                ssems[buf])

    def wait_store(buf):
        for k in range(BPC):
            pltpu.make_async_copy(
                rows_v.at[buf, pl.ds(k * L, L), pl.ds(0, 64)],
                out.at[k],
                ssems[buf]).wait()

    start_gather(0, 0)
    start_gather(1, 1)

    @pl.loop(0, NCH, step=2)
    def _(c):
        for b in range(2):
            cc = c + b
            wait_gather(b)
            add_chunk(cc, b)
            start_store(cc, b)

            @pl.when(cc + 2 < NCH)
            def _():
                wait_store(b)
                start_gather(cc + 2, b)

    wait_store(0)
    wait_store(1)


_sc_call = functools.partial(
    pl.kernel,
    out_type=jax.ShapeDtypeStruct((B, L, D), jnp.float32),
    mesh=plsc.VectorSubcoreMesh(
        core_axis_name="c", subcore_axis_name="s",
        num_cores=NC, num_subcores=NS),
    scratch_types=[
        pltpu.VMEM((IPW, 64), jnp.int32),
        pltpu.VMEM((IPW, 64), jnp.int32),
        pltpu.VMEM((PW + 16,), jnp.int32),
        pltpu.VMEM((VPG, 16), jnp.float32),
        pltpu.VMEM((2, CH, 128), jnp.float32),
        pltpu.SemaphoreType.DMA,
        pltpu.SemaphoreType.DMA,
        pltpu.SemaphoreType.DMA,
        pltpu.SemaphoreType.DMA,
    ],
    compiler_params=pltpu.CompilerParams(use_tc_tiling_on_sc=False),
)(_sc_body)


def kernel(indices, table, W_down, b_down, W_up, b_up):
    idx0 = indices[0]
    adap = _adapter_call(idx0, W_down, b_down.reshape(1, R),
                         W_up, b_up.reshape(1, D), table)
    return _sc_call(table.reshape(V // 2, 128),
                    indices.reshape(N // 64, 64),
                    adap.reshape(VPG, 16))


# speedup vs baseline: 1.0519x; 1.0519x over previous
import functools

import jax
import jax.numpy as jnp
from jax import lax
from jax.experimental import pallas as pl
from jax.experimental.pallas import tpu as pltpu
from jax.experimental.pallas import tpu_sc as plsc

V = 1000000
D = 64
R = 16
B = 16384
L = 20

NC, NS = 2, 16
NW = NC * NS
N = B * L
NB = 256
CPL = B // NB
CPW = L * CPL // NW


def _adap_body(idx0_ref, wd_ref, bd_ref, wu_ref, bu_ref, table_ref,
               out_ref, emb_ref, sem):
    for i in range(L):
        pltpu.make_async_copy(
            table_ref.at[pl.ds(idx0_ref[i], 1)], emb_ref.at[pl.ds(i, 1)], sem
        ).start()
    for i in range(L):
        pltpu.make_async_copy(
            table_ref.at[pl.ds(idx0_ref[i], 1)], emb_ref.at[pl.ds(i, 1)], sem
        ).wait()
    h = emb_ref[...]
    mid = jnp.maximum(
        jnp.dot(h, wd_ref[...], preferred_element_type=jnp.float32)
        + bd_ref[...], 0.0)
    out_ref[...] = (h
                    + jnp.dot(mid, wu_ref[...],
                              preferred_element_type=jnp.float32)
                    + bu_ref[...])


_adapter_call = pl.pallas_call(
    _adap_body,
    out_shape=jax.ShapeDtypeStruct((L, D), jnp.float32),
    in_specs=[
        pl.BlockSpec(memory_space=pltpu.SMEM),
        pl.BlockSpec(memory_space=pltpu.VMEM),
        pl.BlockSpec(memory_space=pltpu.VMEM),
        pl.BlockSpec(memory_space=pltpu.VMEM),
        pl.BlockSpec(memory_space=pltpu.VMEM),
        pl.BlockSpec(memory_space=pltpu.MemorySpace.HBM),
    ],
    out_specs=pl.BlockSpec(memory_space=pltpu.VMEM),
    scratch_shapes=[pltpu.VMEM((L, D), jnp.float32), pltpu.SemaphoreType.DMA],
)


def _sc_body(table2, idxf, adap, out,
             idx_v, idx2_v, adap_v, spl_v, rows_v, tr_v,
             is0, is1, gs0, gs1, ss0, ss1):
    wid = lax.axis_index("s") * NC + lax.axis_index("c")
    cbase = wid * CPW
    l0 = cbase // CPL

    pltpu.sync_copy(adap, adap_v)

    one = jnp.int32(1)
    six = jnp.int32(6)
    iota = lax.iota(jnp.int32, 16)

    for li in range(2):
        l = jnp.minimum(l0 + li, L - 1)
        for d in range(D):
            base = l * D + (d // 16) * 16
            s = adap_v[pl.ds(base, 16)][d % 16]
            spl_v[li * D + d, :] = lax.broadcast(s, (16,))

    isems = (is0, is1)
    gsems = (gs0, gs1)
    ssems = (ss0, ss1)

    def start_idx(c, buf):
        pltpu.async_copy(
            idxf.at[pl.ds((cbase + c) * NB, NB)], idx_v.at[buf], isems[buf])

    def wait_idx(buf):
        pltpu.make_async_copy(
            idxf.at[pl.ds(0, NB)], idx_v.at[buf], isems[buf]).wait()

    def compute_idx2(buf):
        for j in range(2):
            for k in range(8):
                x = idx_v[buf, pl.ds(j * 128 + k * 16, 16)]
                idx2_v[buf, j, pl.ds(k * 16, 16)] = (
                    lax.shift_right_logical(x, one))

    def start_gather(buf):
        for j in range(2):
            pltpu.async_copy(
                table2.at[idx2_v.at[buf, j]],
                rows_v.at[buf, pl.ds(j * 128, 128), :],
                gsems[buf])

    def wait_gather(buf):
        for j in range(2):
            pltpu.make_async_copy(
                table2.at[idx2_v.at[buf, j]],
                rows_v.at[buf, pl.ds(j * 128, 128), :],
                gsems[buf]).wait()

    def add_chunk(c, buf):
        li = (cbase + c) // CPL - l0

        @pl.loop(0, NB // 16)
        def _(blk):
            r0 = blk * 16
            row_ids = iota + r0
            pvec = lax.bitwise_and(idx_v[buf, pl.ds(r0, 16)], one)
            col0 = lax.shift_left(pvec, six)
            for d in range(D):
                cols = col0 + d
                vals = plsc.load_gather(rows_v.at[buf], [row_ids, cols])
                tr_v[buf, d, pl.ds(r0, 16)] = vals + spl_v[li * D + d, :]

    def start_store(c, buf):
        g = cbase + c
        pltpu.async_copy(
            tr_v.at[buf],
            out.at[g // CPL, :, pl.ds((g % CPL) * NB, NB)],
            ssems[buf])

    def wait_store(buf):
        pltpu.make_async_copy(
            tr_v.at[buf], out.at[0, :, pl.ds(0, NB)], ssems[buf]).wait()

    start_idx(0, 0)
    start_idx(1, 1)
    wait_idx(0)
    compute_idx2(0)
    start_gather(0)

    @pl.loop(0, CPW, step=2)
    def _(c):
        for b in range(2):
            cc = c + b
            buf = b
            obuf = 1 - b

            @pl.when(cc + 1 < CPW)
            def _():
                wait_idx(obuf)
                compute_idx2(obuf)

            wait_gather(buf)

            @pl.when(cc + 1 < CPW)
            def _():
                start_gather(obuf)

            @pl.when(cc >= 2)
            def _():
                wait_store(buf)

            add_chunk(cc, buf)
            start_store(cc, buf)

            @pl.when(cc + 2 < CPW)
            def _():
                start_idx(cc + 2, buf)

    wait_store(0)
    wait_store(1)


_sc_call = functools.partial(
    pl.kernel,
    out_type=jax.ShapeDtypeStruct((L, D, B), jnp.float32),
    mesh=plsc.VectorSubcoreMesh(
        core_axis_name="c", subcore_axis_name="s",
        num_cores=NC, num_subcores=NS),
    scratch_types=[
        pltpu.VMEM((2, NB), jnp.int32),
        pltpu.VMEM((2, 2, 128), jnp.int32),
        pltpu.VMEM((L * D,), jnp.float32),
        pltpu.VMEM((2 * D, 16), jnp.float32),
        pltpu.VMEM((2, NB, 128), jnp.float32),
        pltpu.VMEM((2, D, NB), jnp.float32),
        pltpu.SemaphoreType.DMA,
        pltpu.SemaphoreType.DMA,
        pltpu.SemaphoreType.DMA,
        pltpu.SemaphoreType.DMA,
        pltpu.SemaphoreType.DMA,
        pltpu.SemaphoreType.DMA,
    ],
    compiler_params=pltpu.CompilerParams(needs_layout_passes=False),
)(_sc_body)


def kernel(indices, table, W_down, b_down, W_up, b_up):
    idx0 = indices[0]
    adap = _adapter_call(idx0, W_down, b_down.reshape(1, R),
                         W_up, b_up.reshape(1, D), table)
    out_ldb = _sc_call(table.reshape(V // 2, 128),
                       indices.T.reshape(N),
                       adap.reshape(L * D))
    return out_ldb.transpose(2, 0, 1)


# speedup vs baseline: 1.3787x; 1.3106x over previous
import functools

import jax
import jax.numpy as jnp
from jax import lax
from jax.experimental import pallas as pl
from jax.experimental.pallas import tpu as pltpu
from jax.experimental.pallas import tpu_sc as plsc

V = 1000000
D = 64
R = 16
B = 16384
L = 20

NC, NS = 2, 16
NW = NC * NS
N = B * L
NB = 256
CPL = B // NB
CPW = L * CPL // NW


def _adap_body(idx0_ref, wd_ref, bd_ref, wu_ref, bu_ref, table_ref,
               out_ref, emb_ref, sem):
    for i in range(L):
        pltpu.make_async_copy(
            table_ref.at[pl.ds(idx0_ref[i], 1)], emb_ref.at[pl.ds(i, 1)], sem
        ).start()
    for i in range(L):
        pltpu.make_async_copy(
            table_ref.at[pl.ds(idx0_ref[i], 1)], emb_ref.at[pl.ds(i, 1)], sem
        ).wait()
    h = emb_ref[...]
    mid = jnp.maximum(
        jnp.dot(h, wd_ref[...], preferred_element_type=jnp.float32)
        + bd_ref[...], 0.0)
    out_ref[...] = (h
                    + jnp.dot(mid, wu_ref[...],
                              preferred_element_type=jnp.float32)
                    + bu_ref[...])


_adapter_call = pl.pallas_call(
    _adap_body,
    out_shape=jax.ShapeDtypeStruct((L, D), jnp.float32),
    in_specs=[
        pl.BlockSpec(memory_space=pltpu.SMEM),
        pl.BlockSpec(memory_space=pltpu.VMEM),
        pl.BlockSpec(memory_space=pltpu.VMEM),
        pl.BlockSpec(memory_space=pltpu.VMEM),
        pl.BlockSpec(memory_space=pltpu.VMEM),
        pl.BlockSpec(memory_space=pltpu.MemorySpace.HBM),
    ],
    out_specs=pl.BlockSpec(memory_space=pltpu.VMEM),
    scratch_shapes=[pltpu.VMEM((L, D), jnp.float32), pltpu.SemaphoreType.DMA],
)


def _sc_body(table2, idxf, adap, out,
             idx_v, adap_v, spl_v, rows_v, tr_v,
             is0, is1, gs0, gs1, ss0, ss1):
    wid = lax.axis_index("s") * NC + lax.axis_index("c")
    cbase = wid * CPW
    l0 = cbase // CPL

    pltpu.sync_copy(adap, adap_v)

    iota = lax.iota(jnp.int32, 16)

    for li in range(2):
        l = jnp.minimum(l0 + li, L - 1)
        for d in range(D):
            base = l * D + (d // 16) * 16
            s = adap_v[pl.ds(base, 16)][d % 16]
            spl_v[li * D + d, :] = lax.broadcast(s, (16,))

    isems = (is0, is1)
    gsems = (gs0, gs1)
    ssems = (ss0, ss1)

    def start_idx(c, buf):
        pltpu.async_copy(
            idxf.at[pl.ds((cbase + c) * NB, NB)], idx_v.at[buf], isems[buf])

    def wait_idx(buf):
        pltpu.make_async_copy(
            idxf.at[pl.ds(0, NB)], idx_v.at[buf], isems[buf]).wait()

    def start_gather(buf):
        for j in range(2):
            pltpu.async_copy(
                table2.at[idx_v.at[buf, pl.ds(j * 128, 128)]],
                rows_v.at[buf, pl.ds(j * 128, 128), :],
                gsems[buf])

    def wait_gather(buf):
        for j in range(2):
            pltpu.make_async_copy(
                table2.at[idx_v.at[buf, pl.ds(j * 128, 128)]],
                rows_v.at[buf, pl.ds(j * 128, 128), :],
                gsems[buf]).wait()

    def add_chunk(c, buf):
        li = (cbase + c) // CPL - l0

        @plsc.parallel_loop(0, NB // 16)
        def _(blk):
            r0 = blk * 16
            row_ids = iota + r0
            for d in range(D):
                cols = jnp.full((16,), d, jnp.int32)
                vals = plsc.load_gather(rows_v.at[buf], [row_ids, cols])
                tr_v[buf, d, pl.ds(r0, 16)] = vals + spl_v[li * D + d, :]

    def start_store(c, buf):
        g = cbase + c
        pltpu.async_copy(
            tr_v.at[buf],
            out.at[g // CPL, :, pl.ds((g % CPL) * NB, NB)],
            ssems[buf])

    def wait_store(buf):
        pltpu.make_async_copy(
            tr_v.at[buf], out.at[0, :, pl.ds(0, NB)], ssems[buf]).wait()

    start_idx(0, 0)
    start_idx(1, 1)
    wait_idx(0)
    start_gather(0)

    @pl.loop(0, CPW, step=2)
    def _(c):
        for b in range(2):
            cc = c + b
            buf = b
            obuf = 1 - b

            @pl.when(cc + 1 < CPW)
            def _():
                wait_idx(obuf)

            wait_gather(buf)

            @pl.when(cc + 1 < CPW)
            def _():
                start_gather(obuf)

            @pl.when(cc >= 2)
            def _():
                wait_store(buf)

            add_chunk(cc, buf)
            start_store(cc, buf)

            @pl.when(cc + 2 < CPW)
            def _():
                start_idx(cc + 2, buf)

    wait_store(0)
    wait_store(1)


_sc_call = functools.partial(
    pl.kernel,
    out_type=jax.ShapeDtypeStruct((L, D, B), jnp.float32),
    mesh=plsc.VectorSubcoreMesh(
        core_axis_name="c", subcore_axis_name="s",
        num_cores=NC, num_subcores=NS),
    scratch_types=[
        pltpu.VMEM((2, NB), jnp.int32),
        pltpu.VMEM((L * D,), jnp.float32),
        pltpu.VMEM((2 * D, 16), jnp.float32),
        pltpu.VMEM((2, NB, 128), jnp.float32),
        pltpu.VMEM((2, D, NB), jnp.float32),
        pltpu.SemaphoreType.DMA,
        pltpu.SemaphoreType.DMA,
        pltpu.SemaphoreType.DMA,
        pltpu.SemaphoreType.DMA,
        pltpu.SemaphoreType.DMA,
        pltpu.SemaphoreType.DMA,
    ],
    compiler_params=pltpu.CompilerParams(needs_layout_passes=False),
)(_sc_body)


def kernel(indices, table, W_down, b_down, W_up, b_up):
    idx0 = indices[0]
    adap = _adapter_call(idx0, W_down, b_down.reshape(1, R),
                         W_up, b_up.reshape(1, D), table)
    out_ldb = _sc_call(jnp.pad(table, ((0, 0), (0, D))),
                       indices.T.reshape(N),
                       adap.reshape(L * D))
    return out_ldb.transpose(2, 0, 1)
